# Initial kernel scaffold; baseline (speedup 1.0000x reference)
import math

import jax
import jax.numpy as jnp
from jax import lax
from jax.experimental import pallas as pl
from jax.experimental.pallas import tpu as pltpu
from jax.experimental.pallas import tpu_sc as plsc

VOCAB = 1000000
EMBED_SIZE = 32
BATCH = 4096
HIST = 200
SCALE = math.sqrt(EMBED_SIZE)

NC = 2
NS = 16
NW = NC * NS
LANES = 16

B = BATCH * HIST
B_PER_W = B // NW
CHUNK = 1024
N_CHUNKS = B_PER_W // CHUNK


def _body(x_hbm, lut_hbm, out_hbm, idx_v, rows_v, gsem):
    wid = lax.axis_index("s") * NC + lax.axis_index("c")
    base = wid * B_PER_W

    def step(c, carry):
        off = base + c * CHUNK
        pltpu.sync_copy(x_hbm.at[pl.ds(off, CHUNK)], idx_v)
        pltpu.async_copy(lut_hbm.at[idx_v], rows_v, gsem).wait()

        def scale_row(r, rcarry):
            rows_v[r, pl.ds(0, LANES)] = rows_v[r, pl.ds(0, LANES)] * SCALE
            rows_v[r, pl.ds(LANES, LANES)] = (
                rows_v[r, pl.ds(LANES, LANES)] * SCALE
            )
            return rcarry

        lax.fori_loop(0, CHUNK, scale_row, 0, unroll=4)
        pltpu.sync_copy(rows_v, out_hbm.at[pl.ds(off, CHUNK)])
        return carry

    lax.fori_loop(0, N_CHUNKS, step, 0)


@jax.jit
def _lookup(x_flat, lut):
    mesh = plsc.VectorSubcoreMesh(core_axis_name="c", subcore_axis_name="s")
    return pl.kernel(
        _body,
        out_type=jax.ShapeDtypeStruct((B, EMBED_SIZE), jnp.float32),
        mesh=mesh,
        scratch_types=[
            pltpu.VMEM((CHUNK,), jnp.int32),
            pltpu.VMEM((CHUNK, EMBED_SIZE), jnp.float32),
            pltpu.SemaphoreType.DMA,
        ],
    )(x_flat, lut)


def kernel(x, lut):
    out = _lookup(x.reshape(B).astype(jnp.int32), lut)
    return out.reshape(BATCH, HIST, EMBED_SIZE)


# speedup vs baseline: 1.3993x; 1.3993x over previous
import math

import jax
import jax.numpy as jnp
from jax import lax
from jax.experimental import pallas as pl
from jax.experimental.pallas import tpu as pltpu
from jax.experimental.pallas import tpu_sc as plsc

VOCAB = 1000000
EMBED_SIZE = 32
BATCH = 4096
HIST = 200
SCALE = math.sqrt(EMBED_SIZE)

NC = 2
NS = 16
NW = NC * NS
LANES = 16

B = BATCH * HIST
B_PER_W = B // NW
CHUNK = 1024
N_CHUNKS = B_PER_W // CHUNK


def _body(x_hbm, lut_hbm, out_hbm, idx_v, rows_v, gsem):
    wid = lax.axis_index("s") * NC + lax.axis_index("c")
    base = wid * B_PER_W

    def step(c, carry):
        off = base + c * CHUNK
        pltpu.sync_copy(x_hbm.at[pl.ds(off, CHUNK)], idx_v)
        pltpu.async_copy(lut_hbm.at[idx_v], rows_v, gsem).wait()

        def scale_row(r, rcarry):
            rows_v[r, pl.ds(0, LANES)] = rows_v[r, pl.ds(0, LANES)] * SCALE
            rows_v[r, pl.ds(LANES, LANES)] = (
                rows_v[r, pl.ds(LANES, LANES)] * SCALE
            )
            return rcarry

        lax.fori_loop(0, CHUNK, scale_row, 0, unroll=4)
        pltpu.sync_copy(rows_v, out_hbm.at[pl.ds(off, CHUNK)])
        return carry

    lax.fori_loop(0, N_CHUNKS, step, 0)


@jax.jit
def _lookup(x_flat, lut):
    mesh = plsc.VectorSubcoreMesh(core_axis_name="c", subcore_axis_name="s")
    return pl.kernel(
        _body,
        out_type=jax.ShapeDtypeStruct((B, EMBED_SIZE), jnp.float32),
        mesh=mesh,
        scratch_types=[
            pltpu.VMEM((CHUNK,), jnp.int32),
            pltpu.VMEM((CHUNK, EMBED_SIZE), jnp.float32),
            pltpu.SemaphoreType.DMA,
        ],
        compiler_params=pltpu.CompilerParams(use_tc_tiling_on_sc=False),
    )(x_flat, lut)


def kernel(x, lut):
    out = _lookup(x.reshape(B).astype(jnp.int32), lut)
    return out.reshape(BATCH, HIST, EMBED_SIZE)


# speedup vs baseline: 1.4771x; 1.0556x over previous
import math

import jax
import jax.numpy as jnp
from jax import lax
from jax.experimental import pallas as pl
from jax.experimental.pallas import tpu as pltpu
from jax.experimental.pallas import tpu_sc as plsc

VOCAB = 1000000
EMBED_SIZE = 32
BATCH = 4096
HIST = 200
SCALE = math.sqrt(EMBED_SIZE)

NC = 2
NS = 16
NW = NC * NS
LANES = 16

B = BATCH * HIST
B_PER_W = B // NW
NBUF = 4
CHUNK = 640
N_CHUNKS = B_PER_W // CHUNK


def _body(x_hbm, lut_hbm, out_hbm, idx_v, rows, gsems, osems):
    wid = lax.axis_index("s") * NC + lax.axis_index("c")
    base = wid * B_PER_W

    pltpu.sync_copy(x_hbm.at[wid], idx_v)

    for b in range(NBUF):
        pltpu.async_copy(lut_hbm.at[idx_v.at[b]], rows[b], gsems[b])

    def outer(p, carry):
        for b in range(NBUF):
            c = p * NBUF + b
            pltpu.make_async_copy(lut_hbm.at[idx_v.at[c]], rows[b],
                                  gsems[b]).wait()

            def scale_row(r, rc):
                rows[b][r, pl.ds(0, LANES)] = (
                    rows[b][r, pl.ds(0, LANES)] * SCALE)
                rows[b][r, pl.ds(LANES, LANES)] = (
                    rows[b][r, pl.ds(LANES, LANES)] * SCALE)
                return rc

            lax.fori_loop(0, CHUNK, scale_row, 0, unroll=8)

            out_cp = pltpu.make_async_copy(
                rows[b], out_hbm.at[pl.ds(base + c * CHUNK, CHUNK)], osems[b])
            out_cp.start()
            out_cp.wait()

            @pl.when(c + NBUF < N_CHUNKS)
            def _():
                pltpu.async_copy(lut_hbm.at[idx_v.at[c + NBUF]], rows[b],
                                 gsems[b])
        return carry

    lax.fori_loop(0, N_CHUNKS // NBUF, outer, 0)


@jax.jit
def _lookup(x_flat, lut):
    mesh = plsc.VectorSubcoreMesh(core_axis_name="c", subcore_axis_name="s")
    return pl.kernel(
        _body,
        out_type=jax.ShapeDtypeStruct((B, EMBED_SIZE), jnp.float32),
        mesh=mesh,
        scratch_types=[
            pltpu.VMEM((N_CHUNKS, CHUNK), jnp.int32),
            [pltpu.VMEM((CHUNK, EMBED_SIZE), jnp.float32)
             for _ in range(NBUF)],
            [pltpu.SemaphoreType.DMA for _ in range(NBUF)],
            [pltpu.SemaphoreType.DMA for _ in range(NBUF)],
        ],
        compiler_params=pltpu.CompilerParams(use_tc_tiling_on_sc=False),
    )(x_flat, lut)


def kernel(x, lut):
    out = _lookup(x.reshape(NW, N_CHUNKS, CHUNK).astype(jnp.int32), lut)
    return out.reshape(BATCH, HIST, EMBED_SIZE)
